# baseline (speedup 1.0000x reference)
import functools

import jax
import jax.numpy as jnp
from jax import lax
from jax.experimental import pallas as pl
from jax.experimental.pallas import tpu as pltpu
from jax.experimental.pallas import tpu_sc as plsc

N = 50000
D = 64
H = D // 2
E = 800000
EPAD = 819200
C = 128
NSUB = 16
MB = 2560
NBLK = EPAD // NSUB // MB
NCH = MB // C
ROWS_PER_TILE = N // NSUB
WB = 25
RB = 2000
GRID = N // RB

_f32 = jnp.float32


def _spmm_body(x2, rows, cols, vals, out, rvm, vvm, ivm, gbuf0,
               gbuf1, gbuf2, gbuf3, gbuf4, wbuf, acc, gsem, ssem, msem):
    c = lax.axis_index("c")
    s = lax.axis_index("s")

    zeros16 = jnp.zeros((16,), _f32)

    def _zero_wbuf(i, carry):
        wbuf[i, 0:16] = zeros16
        wbuf[i, 16:32] = zeros16
        return carry

    lax.fori_loop(0, WB, _zero_wbuf, 0)

    def _zero_acc(k, carry):
        pltpu.sync_copy(wbuf, acc.at[pl.ds(s * ROWS_PER_TILE + k * WB, WB), :])
        return carry

    lax.fori_loop(0, N // NSUB // WB, _zero_acc, 0)
    plsc.subcore_barrier()

    r_base = s * (EPAD // NSUB // 128)

    def _scale(buf, ch):
        for g in range(C // 16):
            v16 = vvm[ch, pl.ds(g * 16, 16)]
            for j in range(16):
                r = g * 16 + j
                buf[r, 0:16] = buf[r, 0:16] * v16[j]
                buf[r, 16:32] = buf[r, 16:32] * v16[j]

    def _block(blk, carry):
        rb = blk * NCH
        pltpu.async_copy(rows.at[pl.ds(r_base + rb, NCH)], rvm, msem)
        pltpu.async_copy(cols.at[pl.ds(r_base + rb, NCH)], ivm, msem)
        pltpu.async_copy(vals.at[pl.ds(r_base + rb, NCH)], vvm, msem)
        pltpu.make_async_copy(rows.at[pl.ds(r_base + rb, NCH)], rvm, msem).wait()
        pltpu.make_async_copy(cols.at[pl.ds(r_base + rb, NCH)], ivm, msem).wait()
        pltpu.make_async_copy(vals.at[pl.ds(r_base + rb, NCH)], vvm, msem).wait()

        def _xform(k, carry2):
            for g in range(C // 16):
                t = ivm[k, pl.ds(g * 16, 16)]
                ivm[k, pl.ds(g * 16, 16)] = t * 2 + c
            return carry2

        lax.fori_loop(0, NCH, _xform, 0)

        bufs = (gbuf0, gbuf1, gbuf2, gbuf3, gbuf4)
        for b in range(3):
            pltpu.async_copy(x2.at[ivm.at[b]], bufs[b], gsem)

        def _group(i5, carry2):
            for q in range(5):
                i = i5 * 5 + q
                buf = bufs[q]
                nxt = bufs[(q + 3) % 5]
                pltpu.make_async_copy(x2.at[ivm.at[i]], buf, gsem).wait()
                _scale(buf, i)

                if q < 2:
                    @pl.when(i5 >= 1)
                    def _():
                        pltpu.make_async_copy(
                            buf, acc.at[rvm.at[0]], ssem).wait()
                else:
                    pltpu.make_async_copy(
                        buf, acc.at[rvm.at[0]], ssem).wait()

                @pl.when(i + 3 < NCH)
                def _():
                    pltpu.async_copy(x2.at[ivm.at[i + 3]], nxt, gsem)

                pltpu.async_copy(buf, acc.at[rvm.at[i]], ssem, add=True)

            return carry2

        lax.fori_loop(0, NCH // 5, _group, 0)
        pltpu.make_async_copy(gbuf0, acc.at[rvm.at[0]], ssem).wait()
        pltpu.make_async_copy(gbuf0, acc.at[rvm.at[0]], ssem).wait()
        return carry

    lax.fori_loop(0, NBLK, _block, 0)
    plsc.subcore_barrier()

    def _wb(k, carry):
        r0 = s * ROWS_PER_TILE + k * WB
        pltpu.sync_copy(acc.at[pl.ds(r0, WB), :], wbuf)
        pltpu.sync_copy(wbuf, out.at[pl.ds(r0, WB), c, :])
        return carry

    lax.fori_loop(0, N // NSUB // WB, _wb, 0)


@jax.jit
def _spmm(x, rows, cols, vals):
    mesh = plsc.VectorSubcoreMesh(core_axis_name="c", subcore_axis_name="s")
    x2 = x.reshape(2 * N, H)
    rows = rows.reshape(EPAD // 128, 128)
    cols = cols.reshape(EPAD // 128, 128)
    vals = vals.reshape(EPAD // 128, 128)
    out = pl.kernel(
        _spmm_body,
        out_type=jax.ShapeDtypeStruct((N, 2, H), _f32),
        mesh=mesh,
        compiler_params=pltpu.CompilerParams(use_tc_tiling_on_sc=False),
        scratch_types=[
            pltpu.VMEM((NCH, C), jnp.int32),
            pltpu.VMEM((NCH, C), _f32),
            pltpu.VMEM((NCH, C), jnp.int32),
            pltpu.VMEM((C, H), _f32),
            pltpu.VMEM((C, H), _f32),
            pltpu.VMEM((C, H), _f32),
            pltpu.VMEM((C, H), _f32),
            pltpu.VMEM((C, H), _f32),
            pltpu.VMEM((WB, H), _f32),
            pltpu.VMEM_SHARED((N, H), _f32),
            pltpu.SemaphoreType.DMA,
            pltpu.SemaphoreType.DMA,
            pltpu.SemaphoreType.DMA,
        ],
    )(x2, rows, cols, vals)
    return out.reshape(N, D)


def _gate_k(em_ref, w_ref, b_ref, o0, o1, o2, o3):
    em = em_ref[...]
    for i, o in enumerate((o0, o1, o2, o3)):
        z = jnp.dot(em, w_ref[i], preferred_element_type=_f32) + b_ref[i][None, :]
        o[...] = em * jax.nn.sigmoid(z)


def _attn_mix(u0, u1, u2, am_ref, aa_ref):
    ws = []
    for u in (u0, u1, u2):
        t = jnp.dot(u, am_ref[...], preferred_element_type=_f32)
        ws.append(jnp.sum(aa_ref[...] * t, axis=1))
    m = jnp.maximum(jnp.maximum(ws[0], ws[1]), ws[2])
    es = [jnp.exp(w - m) for w in ws]
    tot = es[0] + es[1] + es[2]
    mixed = es[0][:, None] * u0 + es[1][:, None] * u1 + es[2][:, None] * u2
    return mixed / tot[:, None]


def _mix_k(u0, u1, u2, us, am_ref, aa_ref, mo):
    mixed = _attn_mix(u0[...], u1[...], u2[...], am_ref, aa_ref)
    mo[...] = (mixed + us[...]) * 0.5


def _normacc_k(a0, x0, a1, x1, a2, x2, a3, x3, a4, x4, o0, o1, o2, o3, o4):
    for a, x, o in ((a0, x0, o0), (a1, x1, o1), (a2, x2, o2), (a3, x3, o3),
                    (a4, x4, o4)):
        xx = x[...]
        n = jnp.sqrt(jnp.sum(xx * xx, axis=1, keepdims=True))
        o[...] = a[...] + xx / jnp.maximum(n, 1e-12)


def _final_k(a0, a1, a2, aS, am_ref, aa_ref, sw_ref, sb_ref, fu, s0, s1, s2):
    mixed = _attn_mix(a0[...], a1[...], a2[...], am_ref, aa_ref)
    f = mixed + aS[...] * 0.5
    fu[...] = f
    for i, o in enumerate((s0, s1, s2)):
        z = jnp.dot(f, sw_ref[i], preferred_element_type=_f32) + sb_ref[i][None, :]
        o[...] = f * jax.nn.sigmoid(z)


_row_spec = pl.BlockSpec((RB, D), lambda i: (i, 0))


def _full_spec(shape):
    return pl.BlockSpec(shape, lambda i: tuple(0 for _ in shape))


def _rows_out(n):
    return tuple(jax.ShapeDtypeStruct((N, D), _f32) for _ in range(n))


@jax.jit
def _gate(em, w, b):
    return pl.pallas_call(
        _gate_k,
        grid=(GRID,),
        in_specs=[_row_spec, _full_spec((4, D, D)), _full_spec((4, D))],
        out_specs=(_row_spec,) * 4,
        out_shape=_rows_out(4),
    )(em, w, b)


@jax.jit
def _mix(u0, u1, u2, us, am, aa):
    return pl.pallas_call(
        _mix_k,
        grid=(GRID,),
        in_specs=[_row_spec] * 4 + [_full_spec((D, D)), _full_spec((1, D))],
        out_specs=_row_spec,
        out_shape=jax.ShapeDtypeStruct((N, D), _f32),
    )(u0, u1, u2, us, am, aa)


@jax.jit
def _normacc(a0, x0, a1, x1, a2, x2, a3, x3, a4, x4):
    return pl.pallas_call(
        _normacc_k,
        grid=(GRID,),
        in_specs=[_row_spec] * 10,
        out_specs=(_row_spec,) * 5,
        out_shape=_rows_out(5),
    )(a0, x0, a1, x1, a2, x2, a3, x3, a4, x4)


@jax.jit
def _final(a0, a1, a2, aS, am, aa, sw, sb):
    return pl.pallas_call(
        _final_k,
        grid=(GRID,),
        in_specs=[_row_spec] * 4
        + [_full_spec((D, D)), _full_spec((1, D)), _full_spec((4, D, D)),
           _full_spec((4, D))],
        out_specs=(_row_spec,) * 4,
        out_shape=_rows_out(4),
    )(a0, a1, a2, aS, am, aa, sw, sb)


def kernel(user_emb, item_emb, gating_w, gating_b, sgating_w, sgating_b,
           att_mat, att_agg, hs_rows, hs_cols, hs_vals, hj_rows, hj_cols,
           hj_vals, hp_rows, hp_cols, hp_vals, inter_rows, inter_cols,
           inter_vals):
    i32 = jnp.int32

    _spread = (jnp.arange(EPAD - E, dtype=i32) * 41) % N

    def _pad(r, cc, v):
        zf = jnp.zeros((EPAD - E,), _f32)
        return (jnp.concatenate([r.astype(i32), _spread]),
                jnp.concatenate([cc.astype(i32), _spread]),
                jnp.concatenate([v, zf]))

    hs_rows, hs_cols, hs_vals = _pad(hs_rows, hs_cols, hs_vals)
    hj_rows, hj_cols, hj_vals = _pad(hj_rows, hj_cols, hj_vals)
    hp_rows, hp_cols, hp_vals = _pad(hp_rows, hp_cols, hp_vals)
    inter_rows, inter_cols, inter_vals = _pad(inter_rows, inter_cols,
                                              inter_vals)

    u0, u1, u2, us = _gate(user_emb, gating_w, gating_b)
    acc0, acc1, acc2, accS, accI = u0, u1, u2, us, item_emb
    it = item_emb
    for _ in range(2):
        mixed = _mix(u0, u1, u2, us, att_mat, att_agg)
        u0n = _spmm(u0, hs_rows, hs_cols, hs_vals)
        u1n = _spmm(u1, hj_rows, hj_cols, hj_vals)
        u2n = _spmm(u2, hp_rows, hp_cols, hp_vals)
        itn = _spmm(mixed, inter_cols, inter_rows, inter_vals)
        usn = _spmm(it, inter_rows, inter_cols, inter_vals)
        acc0, acc1, acc2, accS, accI = _normacc(
            acc0, u0n, acc1, u1n, acc2, u2n, accS, usn, accI, itn)
        u0, u1, u2, us, it = u0n, u1n, u2n, usn, itn
    fu, s0, s1, s2 = _final(acc0, acc1, acc2, accS, att_mat, att_agg,
                            sgating_w, sgating_b)
    return (fu, accI, (s0, s1, s2))


# speedup vs baseline: 1.2775x; 1.2775x over previous
import functools

import jax
import jax.numpy as jnp
from jax import lax
from jax.experimental import pallas as pl
from jax.experimental.pallas import tpu as pltpu
from jax.experimental.pallas import tpu_sc as plsc

N = 50000
D = 64
H = D // 2
E = 800000
EPAD = 819200
C = 128
NSUB = 16
MB = 2048
NBLK = EPAD // NSUB // MB
NCH = MB // C
ROWS_PER_TILE = N // NSUB
WB = 125
RB = 2000
GRID = N // RB

_f32 = jnp.float32


def _spmm_body(x2, rows, cols, vals, out, rvm, cvm, vvm, ivm, gbuf0,
               gbuf1, gbuf2, gbuf3, wbuf, acc, gsem, ssem, msem):
    c = lax.axis_index("c")
    s = lax.axis_index("s")

    zeros16 = jnp.zeros((16,), _f32)

    def _zero_wbuf(i, carry):
        wbuf[i, 0:16] = zeros16
        wbuf[i, 16:32] = zeros16
        return carry

    lax.fori_loop(0, WB, _zero_wbuf, 0)

    def _zero_acc(k, carry):
        pltpu.sync_copy(wbuf, acc.at[pl.ds(s * ROWS_PER_TILE + k * WB, WB), :])
        return carry

    lax.fori_loop(0, N // NSUB // WB, _zero_acc, 0)
    plsc.subcore_barrier()

    r_base = s * (EPAD // NSUB // 128)

    def _scale(buf, ch):
        for g in range(C // 16):
            v16 = vvm[ch, pl.ds(g * 16, 16)]
            for j in range(16):
                r = g * 16 + j
                buf[r, 0:16] = buf[r, 0:16] * v16[j]
                buf[r, 16:32] = buf[r, 16:32] * v16[j]

    def _block(blk, carry):
        rb = blk * NCH
        pltpu.async_copy(rows.at[pl.ds(r_base + rb, NCH)], rvm, msem)
        pltpu.async_copy(cols.at[pl.ds(r_base + rb, NCH)], cvm, msem)
        pltpu.async_copy(vals.at[pl.ds(r_base + rb, NCH)], vvm, msem)
        pltpu.make_async_copy(rows.at[pl.ds(r_base + rb, NCH)], rvm, msem).wait()
        pltpu.make_async_copy(cols.at[pl.ds(r_base + rb, NCH)], cvm, msem).wait()
        pltpu.make_async_copy(vals.at[pl.ds(r_base + rb, NCH)], vvm, msem).wait()

        def _xform(k, carry2):
            for g in range(C // 16):
                t = cvm[k, pl.ds(g * 16, 16)]
                ivm[k, pl.ds(g * 16, 16)] = t * 2 + c
            return carry2

        lax.fori_loop(0, NCH, _xform, 0)

        bufs = (gbuf0, gbuf1, gbuf2, gbuf3)
        for b in range(3):
            pltpu.async_copy(x2.at[ivm.at[b]], bufs[b], gsem)

        def _group(i4, carry2):
            for q in range(4):
                i = i4 * 4 + q
                buf = bufs[q]
                nxt = bufs[(q + 3) % 4]
                pltpu.make_async_copy(x2.at[ivm.at[i]], buf, gsem).wait()
                _scale(buf, i)

                if q == 0:
                    @pl.when(i4 >= 1)
                    def _():
                        pltpu.make_async_copy(
                            buf, acc.at[rvm.at[0]], ssem).wait()
                else:
                    pltpu.make_async_copy(
                        buf, acc.at[rvm.at[0]], ssem).wait()

                @pl.when(i + 3 < NCH)
                def _():
                    pltpu.async_copy(x2.at[ivm.at[i + 3]], nxt, gsem)

                pltpu.async_copy(buf, acc.at[rvm.at[i]], ssem, add=True)

            return carry2

        lax.fori_loop(0, NCH // 4, _group, 0)
        pltpu.make_async_copy(gbuf0, acc.at[rvm.at[0]], ssem).wait()
        return carry

    lax.fori_loop(0, NBLK, _block, 0)
    plsc.subcore_barrier()

    def _wb(k, carry):
        r0 = s * ROWS_PER_TILE + k * WB
        pltpu.sync_copy(acc.at[pl.ds(r0, WB), :], wbuf)
        pltpu.sync_copy(wbuf, out.at[pl.ds(r0, WB), c, :])
        return carry

    lax.fori_loop(0, N // NSUB // WB, _wb, 0)


@jax.jit
def _spmm(x, rows, cols, vals):
    mesh = plsc.VectorSubcoreMesh(core_axis_name="c", subcore_axis_name="s")
    x2 = x.reshape(2 * N, H)
    rows = rows.reshape(EPAD // 128, 128)
    cols = cols.reshape(EPAD // 128, 128)
    vals = vals.reshape(EPAD // 128, 128)
    out = pl.kernel(
        _spmm_body,
        out_type=jax.ShapeDtypeStruct((N, 2, H), _f32),
        mesh=mesh,
        compiler_params=pltpu.CompilerParams(use_tc_tiling_on_sc=False),
        scratch_types=[
            pltpu.VMEM((NCH, C), jnp.int32),
            pltpu.VMEM((NCH, C), jnp.int32),
            pltpu.VMEM((NCH, C), _f32),
            pltpu.VMEM((NCH, C), jnp.int32),
            pltpu.VMEM((C, H), _f32),
            pltpu.VMEM((C, H), _f32),
            pltpu.VMEM((C, H), _f32),
            pltpu.VMEM((C, H), _f32),
            pltpu.VMEM((WB, H), _f32),
            pltpu.VMEM_SHARED((N, H), _f32),
            pltpu.SemaphoreType.DMA,
            pltpu.SemaphoreType.DMA,
            pltpu.SemaphoreType.DMA,
        ],
    )(x2, rows, cols, vals)
    return out.reshape(N, D)


def _gate_k(em_ref, w_ref, b_ref, o0, o1, o2, o3):
    em = em_ref[...]
    for i, o in enumerate((o0, o1, o2, o3)):
        z = jnp.dot(em, w_ref[i], preferred_element_type=_f32) + b_ref[i][None, :]
        o[...] = em * jax.nn.sigmoid(z)


def _attn_mix(u0, u1, u2, am_ref, aa_ref):
    ws = []
    for u in (u0, u1, u2):
        t = jnp.dot(u, am_ref[...], preferred_element_type=_f32)
        ws.append(jnp.sum(aa_ref[...] * t, axis=1))
    m = jnp.maximum(jnp.maximum(ws[0], ws[1]), ws[2])
    es = [jnp.exp(w - m) for w in ws]
    tot = es[0] + es[1] + es[2]
    mixed = es[0][:, None] * u0 + es[1][:, None] * u1 + es[2][:, None] * u2
    return mixed / tot[:, None]


def _mix_k(u0, u1, u2, us, am_ref, aa_ref, mo):
    mixed = _attn_mix(u0[...], u1[...], u2[...], am_ref, aa_ref)
    mo[...] = (mixed + us[...]) * 0.5


def _normacc_k(a0, x0, a1, x1, a2, x2, a3, x3, a4, x4, o0, o1, o2, o3, o4):
    for a, x, o in ((a0, x0, o0), (a1, x1, o1), (a2, x2, o2), (a3, x3, o3),
                    (a4, x4, o4)):
        xx = x[...]
        n = jnp.sqrt(jnp.sum(xx * xx, axis=1, keepdims=True))
        o[...] = a[...] + xx / jnp.maximum(n, 1e-12)


def _final_k(a0, a1, a2, aS, am_ref, aa_ref, sw_ref, sb_ref, fu, s0, s1, s2):
    mixed = _attn_mix(a0[...], a1[...], a2[...], am_ref, aa_ref)
    f = mixed + aS[...] * 0.5
    fu[...] = f
    for i, o in enumerate((s0, s1, s2)):
        z = jnp.dot(f, sw_ref[i], preferred_element_type=_f32) + sb_ref[i][None, :]
        o[...] = f * jax.nn.sigmoid(z)


_row_spec = pl.BlockSpec((RB, D), lambda i: (i, 0))


def _full_spec(shape):
    return pl.BlockSpec(shape, lambda i: tuple(0 for _ in shape))


def _rows_out(n):
    return tuple(jax.ShapeDtypeStruct((N, D), _f32) for _ in range(n))


@jax.jit
def _gate(em, w, b):
    return pl.pallas_call(
        _gate_k,
        grid=(GRID,),
        in_specs=[_row_spec, _full_spec((4, D, D)), _full_spec((4, D))],
        out_specs=(_row_spec,) * 4,
        out_shape=_rows_out(4),
    )(em, w, b)


@jax.jit
def _mix(u0, u1, u2, us, am, aa):
    return pl.pallas_call(
        _mix_k,
        grid=(GRID,),
        in_specs=[_row_spec] * 4 + [_full_spec((D, D)), _full_spec((1, D))],
        out_specs=_row_spec,
        out_shape=jax.ShapeDtypeStruct((N, D), _f32),
    )(u0, u1, u2, us, am, aa)


@jax.jit
def _normacc(a0, x0, a1, x1, a2, x2, a3, x3, a4, x4):
    return pl.pallas_call(
        _normacc_k,
        grid=(GRID,),
        in_specs=[_row_spec] * 10,
        out_specs=(_row_spec,) * 5,
        out_shape=_rows_out(5),
    )(a0, x0, a1, x1, a2, x2, a3, x3, a4, x4)


@jax.jit
def _final(a0, a1, a2, aS, am, aa, sw, sb):
    return pl.pallas_call(
        _final_k,
        grid=(GRID,),
        in_specs=[_row_spec] * 4
        + [_full_spec((D, D)), _full_spec((1, D)), _full_spec((4, D, D)),
           _full_spec((4, D))],
        out_specs=(_row_spec,) * 4,
        out_shape=_rows_out(4),
    )(a0, a1, a2, aS, am, aa, sw, sb)


def kernel(user_emb, item_emb, gating_w, gating_b, sgating_w, sgating_b,
           att_mat, att_agg, hs_rows, hs_cols, hs_vals, hj_rows, hj_cols,
           hj_vals, hp_rows, hp_cols, hp_vals, inter_rows, inter_cols,
           inter_vals):
    i32 = jnp.int32

    _spread = (jnp.arange(EPAD - E, dtype=i32) * 41) % N

    def _pad(r, cc, v):
        zf = jnp.zeros((EPAD - E,), _f32)
        return (jnp.concatenate([r.astype(i32), _spread]),
                jnp.concatenate([cc.astype(i32), _spread]),
                jnp.concatenate([v, zf]))

    hs_rows, hs_cols, hs_vals = _pad(hs_rows, hs_cols, hs_vals)
    hj_rows, hj_cols, hj_vals = _pad(hj_rows, hj_cols, hj_vals)
    hp_rows, hp_cols, hp_vals = _pad(hp_rows, hp_cols, hp_vals)
    inter_rows, inter_cols, inter_vals = _pad(inter_rows, inter_cols,
                                              inter_vals)

    u0, u1, u2, us = _gate(user_emb, gating_w, gating_b)
    acc0, acc1, acc2, accS, accI = u0, u1, u2, us, item_emb
    it = item_emb
    for _ in range(2):
        mixed = _mix(u0, u1, u2, us, att_mat, att_agg)
        u0n = _spmm(u0, hs_rows, hs_cols, hs_vals)
        u1n = _spmm(u1, hj_rows, hj_cols, hj_vals)
        u2n = _spmm(u2, hp_rows, hp_cols, hp_vals)
        itn = _spmm(mixed, inter_cols, inter_rows, inter_vals)
        usn = _spmm(it, inter_rows, inter_cols, inter_vals)
        acc0, acc1, acc2, accS, accI = _normacc(
            acc0, u0n, acc1, u1n, acc2, u2n, accS, usn, accI, itn)
        u0, u1, u2, us, it = u0n, u1n, u2n, usn, itn
    fu, s0, s1, s2 = _final(acc0, acc1, acc2, accS, att_mat, att_agg,
                            sgating_w, sgating_b)
    return (fu, accI, (s0, s1, s2))
